# Initial kernel scaffold; baseline (speedup 1.0000x reference)
import jax
import jax.numpy as jnp
from jax.experimental import pallas as pl


def kernel(node_type_ids, edge_index, node_emb, type_emb, W1, b1, W2, b2):
    raise NotImplementedError("write your pallas kernel here")


# speedup vs baseline: 7.2025x; 7.2025x over previous
import functools

import jax
import jax.numpy as jnp
from jax import lax
from jax.experimental import pallas as pl
from jax.experimental.pallas import tpu as pltpu
from jax.experimental.pallas import tpu_sc as plsc

N = 10000
E = 320000
NT = 16
H = 128

NC = 2
NS = 16
NW = NC * NS
EPW = E // NW
B = 80
CH = EPW // B
RPW0 = 640
RPW1 = 624

BN = 1000
GRID = N // BN

_mesh = plsc.VectorSubcoreMesh(core_axis_name="c", subcore_axis_name="s")


@functools.partial(
    pl.kernel,
    mesh=_mesh,
    out_type=jax.ShapeDtypeStruct((NC, N, H), jnp.float32),
    scratch_types=[
        pltpu.VMEM((CH, B), jnp.int32),
        pltpu.VMEM((CH, B), jnp.int32),
        pltpu.VMEM((B, H), jnp.float32),
        pltpu.VMEM_SHARED((N, H), jnp.float32),
        pltpu.SemaphoreType.DMA,
    ],
)
def _sc_agg(col_hbm, row_hbm, table_hbm, zeros_hbm, out_hbm,
            col_v, row_v, rows_v, acc_sh, sem):
    c = lax.axis_index("c")
    s = lax.axis_index("s")
    wid = c * NS + s
    base1 = pl.multiple_of(16 + s * RPW1, 8)

    @pl.when(s == 0)
    def _():
        pltpu.sync_copy(zeros_hbm, acc_sh.at[pl.ds(0, RPW0)])

    @pl.when(s != 0)
    def _():
        pltpu.sync_copy(zeros_hbm.at[pl.ds(0, RPW1)],
                        acc_sh.at[pl.ds(base1, RPW1)])

    plsc.subcore_barrier()
    pltpu.sync_copy(col_hbm.at[wid], col_v)
    pltpu.sync_copy(row_hbm.at[wid], row_v)

    def body(j, carry):
        pltpu.async_copy(table_hbm.at[col_v.at[j]], rows_v, sem).wait()
        pltpu.sync_copy(rows_v, acc_sh.at[row_v.at[j]], add=True)
        return carry

    lax.fori_loop(0, CH, body, 0)
    plsc.subcore_barrier()

    @pl.when(s == 0)
    def _():
        pltpu.sync_copy(acc_sh.at[pl.ds(0, RPW0)],
                        out_hbm.at[c, pl.ds(0, RPW0)])

    @pl.when(s != 0)
    def _():
        pltpu.sync_copy(acc_sh.at[pl.ds(base1, RPW1)],
                        out_hbm.at[c, pl.ds(base1, RPW1)])


def _embed_body(ids_ref, nemb_ref, temb_ref, o_ref):
    ids = ids_ref[0, 0, :]
    onehot = (ids[:, None]
              == lax.broadcasted_iota(jnp.int32, (1, NT), 1)).astype(jnp.float32)
    o_ref[...] = nemb_ref[...] + jnp.dot(
        onehot, temb_ref[...], preferred_element_type=jnp.float32)


def _mlp_body(parts_ref, w1_ref, b1_ref, w2_ref, o_ref):
    p = parts_ref[0] + parts_ref[1]
    h = jnp.maximum(
        jnp.dot(p, w1_ref[...], preferred_element_type=jnp.float32)
        + b1_ref[...], 0.0)
    o_ref[...] = jnp.dot(h, w2_ref[...], preferred_element_type=jnp.float32)


def _final_body(parts_ref, b2_ref, o_ref):
    o_ref[...] = parts_ref[0] + parts_ref[1] + b2_ref[...]


_embed = pl.pallas_call(
    _embed_body,
    grid=(GRID,),
    in_specs=[
        pl.BlockSpec((1, 1, BN), lambda i: (i, 0, 0)),
        pl.BlockSpec((BN, H), lambda i: (i, 0)),
        pl.BlockSpec((NT, H), lambda i: (0, 0)),
    ],
    out_specs=pl.BlockSpec((BN, H), lambda i: (i, 0)),
    out_shape=jax.ShapeDtypeStruct((N, H), jnp.float32),
)

_mlp = pl.pallas_call(
    _mlp_body,
    grid=(GRID,),
    in_specs=[
        pl.BlockSpec((NC, BN, H), lambda i: (0, i, 0)),
        pl.BlockSpec((H, H), lambda i: (0, 0)),
        pl.BlockSpec((1, H), lambda i: (0, 0)),
        pl.BlockSpec((H, H), lambda i: (0, 0)),
    ],
    out_specs=pl.BlockSpec((BN, H), lambda i: (i, 0)),
    out_shape=jax.ShapeDtypeStruct((N, H), jnp.float32),
)

_final = pl.pallas_call(
    _final_body,
    grid=(GRID,),
    in_specs=[
        pl.BlockSpec((NC, BN, H), lambda i: (0, i, 0)),
        pl.BlockSpec((1, H), lambda i: (0, 0)),
    ],
    out_specs=pl.BlockSpec((BN, H), lambda i: (i, 0)),
    out_shape=jax.ShapeDtypeStruct((N, H), jnp.float32),
)


def kernel(node_type_ids, edge_index, node_emb, type_emb, W1, b1, W2, b2):
    col = edge_index[1].reshape(NW, CH, B)
    row = edge_index[0].reshape(NW, CH, B)
    zeros = jnp.zeros((RPW0, H), jnp.float32)
    ids3 = node_type_ids.reshape(GRID, 1, BN).astype(jnp.int32)

    x = _embed(ids3, node_emb, type_emb)
    p1 = _sc_agg(col, row, x, zeros)
    y = _mlp(p1, W1, b1.reshape(1, H), W2)
    p2 = _sc_agg(col, row, y, zeros)
    return _final(p2, b2.reshape(1, H))
